# initial kernel scaffold (unmeasured)
import jax
import jax.numpy as jnp
from jax import lax
from jax.experimental import pallas as pl
from jax.experimental.pallas import tpu as pltpu


def kernel(
    u,
):
    def body(*refs):
        pass

    out_shape = jax.ShapeDtypeStruct(..., jnp.float32)
    return pl.pallas_call(body, out_shape=out_shape)(...)


# baseline (device time: 11225 ns/iter reference)
import jax
import jax.numpy as jnp
from jax import lax
from jax.experimental import pallas as pl
from jax.experimental.pallas import tpu as pltpu


def kernel(u):
    sx, sy, sz = u.shape

    def body(
        u_ref, out_ref,
        fx_ref, fy_ref, fz_ref,
        hx_ref, hy_ref, hz_ref,
        send_sems, recv_sems,
    ):
        mx = lax.axis_index("x")
        my = lax.axis_index("y")
        mz = lax.axis_index("z")

        nbr_x = (1 - mx, my, mz)
        nbr_y = (mx, 1 - my, mz)
        nbr_z = (mx, my, 1 - mz)

        barrier = pltpu.get_barrier_semaphore()
        for nbr in (nbr_x, nbr_y, nbr_z):
            pl.semaphore_signal(
                barrier, inc=1, device_id=nbr,
                device_id_type=pl.DeviceIdType.MESH,
            )
        pl.semaphore_wait(barrier, 3)

        x = u_ref[:, :, :]

        @pl.when(mx == 0)
        def _():
            fx_ref[:, :] = x[sx - 1, :, :]

        @pl.when(mx == 1)
        def _():
            fx_ref[:, :] = x[0, :, :]

        @pl.when(my == 0)
        def _():
            fy_ref[:, :] = x[:, sy - 1, :]

        @pl.when(my == 1)
        def _():
            fy_ref[:, :] = x[:, 0, :]

        @pl.when(mz == 0)
        def _():
            fz_ref[:, :] = x[:, :, sz - 1]

        @pl.when(mz == 1)
        def _():
            fz_ref[:, :] = x[:, :, 0]

        rdma_x = pltpu.make_async_remote_copy(
            src_ref=fx_ref, dst_ref=hx_ref,
            send_sem=send_sems.at[0], recv_sem=recv_sems.at[0],
            device_id=nbr_x, device_id_type=pl.DeviceIdType.MESH,
        )
        rdma_y = pltpu.make_async_remote_copy(
            src_ref=fy_ref, dst_ref=hy_ref,
            send_sem=send_sems.at[1], recv_sem=recv_sems.at[1],
            device_id=nbr_y, device_id_type=pl.DeviceIdType.MESH,
        )
        rdma_z = pltpu.make_async_remote_copy(
            src_ref=fz_ref, dst_ref=hz_ref,
            send_sem=send_sems.at[2], recv_sem=recv_sems.at[2],
            device_id=nbr_z, device_id_type=pl.DeviceIdType.MESH,
        )
        rdma_x.start()
        rdma_y.start()
        rdma_z.start()
        rdma_x.wait()
        rdma_y.wait()
        rdma_z.wait()

        hx = hx_ref[:, :][None, :, :]
        hy = hy_ref[:, :][:, None, :]
        hz = hz_ref[:, :][:, :, None]

        um_x = jnp.concatenate([hx, x[:-1, :, :]], axis=0)
        up_x = jnp.concatenate([x[1:, :, :], hx], axis=0)
        um_y = jnp.concatenate([hy, x[:, :-1, :]], axis=1)
        up_y = jnp.concatenate([x[:, 1:, :], hy], axis=1)
        um_z = jnp.concatenate([hz, x[:, :, :-1]], axis=2)
        up_z = jnp.concatenate([x[:, :, 1:], hz], axis=2)

        out_ref[:, :, :] = (
            um_x + up_x + um_y + up_y + um_z + up_z - 6.0 * x
        )

        @pl.when(mx == 0)
        def _():
            out_ref[pl.ds(0, 1), :, :] = jnp.zeros((1, sy, sz), jnp.float32)

        @pl.when(mx == 1)
        def _():
            out_ref[pl.ds(sx - 1, 1), :, :] = jnp.zeros((1, sy, sz), jnp.float32)

        @pl.when(my == 0)
        def _():
            out_ref[:, pl.ds(0, 1), :] = jnp.zeros((sx, 1, sz), jnp.float32)

        @pl.when(my == 1)
        def _():
            out_ref[:, pl.ds(sy - 1, 1), :] = jnp.zeros((sx, 1, sz), jnp.float32)

        @pl.when(mz == 0)
        def _():
            out_ref[:, :, pl.ds(0, 1)] = jnp.zeros((sx, sy, 1), jnp.float32)

        @pl.when(mz == 1)
        def _():
            out_ref[:, :, pl.ds(sz - 1, 1)] = jnp.zeros((sx, sy, 1), jnp.float32)

    return pl.pallas_call(
        body,
        out_shape=jax.ShapeDtypeStruct((sx, sy, sz), jnp.float32),
        in_specs=[pl.BlockSpec(memory_space=pltpu.VMEM)],
        out_specs=pl.BlockSpec(memory_space=pltpu.VMEM),
        scratch_shapes=[
            pltpu.VMEM((sy, sz), jnp.float32),
            pltpu.VMEM((sx, sz), jnp.float32),
            pltpu.VMEM((sx, sy), jnp.float32),
            pltpu.VMEM((sy, sz), jnp.float32),
            pltpu.VMEM((sx, sz), jnp.float32),
            pltpu.VMEM((sx, sy), jnp.float32),
            pltpu.SemaphoreType.DMA((3,)),
            pltpu.SemaphoreType.DMA((3,)),
        ],
        compiler_params=pltpu.CompilerParams(collective_id=0),
    )(u)


# device time: 10734 ns/iter; 1.0457x vs baseline; 1.0457x over previous
import jax
import jax.numpy as jnp
from jax import lax
from jax.experimental import pallas as pl
from jax.experimental.pallas import tpu as pltpu


def kernel(u):
    sx, sy, sz = u.shape

    def body(
        u_ref, out_ref,
        fx_ref, fy_ref, fz_ref,
        hx_ref, hy_ref, hz_ref,
        send_sems, recv_sems,
    ):
        mx = lax.axis_index("x")
        my = lax.axis_index("y")
        mz = lax.axis_index("z")

        nbr_x = (1 - mx, my, mz)
        nbr_y = (mx, 1 - my, mz)
        nbr_z = (mx, my, 1 - mz)

        x = u_ref[:, :, :]

        @pl.when(mx == 0)
        def _():
            fx_ref[:, :] = x[sx - 1, :, :]

        @pl.when(mx == 1)
        def _():
            fx_ref[:, :] = x[0, :, :]

        @pl.when(my == 0)
        def _():
            fy_ref[:, :] = x[:, sy - 1, :]

        @pl.when(my == 1)
        def _():
            fy_ref[:, :] = x[:, 0, :]

        @pl.when(mz == 0)
        def _():
            fz_ref[:, :] = x[:, :, sz - 1]

        @pl.when(mz == 1)
        def _():
            fz_ref[:, :] = x[:, :, 0]

        barrier = pltpu.get_barrier_semaphore()
        for nbr in (nbr_x, nbr_y, nbr_z):
            pl.semaphore_signal(
                barrier, inc=1, device_id=nbr,
                device_id_type=pl.DeviceIdType.MESH,
            )
        pl.semaphore_wait(barrier, 3)

        rdma_x = pltpu.make_async_remote_copy(
            src_ref=fx_ref, dst_ref=hx_ref,
            send_sem=send_sems.at[0], recv_sem=recv_sems.at[0],
            device_id=nbr_x, device_id_type=pl.DeviceIdType.MESH,
        )
        rdma_y = pltpu.make_async_remote_copy(
            src_ref=fy_ref, dst_ref=hy_ref,
            send_sem=send_sems.at[1], recv_sem=recv_sems.at[1],
            device_id=nbr_y, device_id_type=pl.DeviceIdType.MESH,
        )
        rdma_z = pltpu.make_async_remote_copy(
            src_ref=fz_ref, dst_ref=hz_ref,
            send_sem=send_sems.at[2], recv_sem=recv_sems.at[2],
            device_id=nbr_z, device_id_type=pl.DeviceIdType.MESH,
        )
        rdma_x.start()
        rdma_y.start()
        rdma_z.start()

        zx = jnp.zeros((1, sy, sz), jnp.float32)
        zy = jnp.zeros((sx, 1, sz), jnp.float32)
        zz = jnp.zeros((sx, sy, 1), jnp.float32)
        um_x = jnp.concatenate([zx, x[:-1, :, :]], axis=0)
        up_x = jnp.concatenate([x[1:, :, :], zx], axis=0)
        um_y = jnp.concatenate([zy, x[:, :-1, :]], axis=1)
        up_y = jnp.concatenate([x[:, 1:, :], zy], axis=1)
        um_z = jnp.concatenate([zz, x[:, :, :-1]], axis=2)
        up_z = jnp.concatenate([x[:, :, 1:], zz], axis=2)

        out_ref[:, :, :] = (
            um_x + up_x + um_y + up_y + um_z + up_z - 6.0 * x
        )

        rdma_x.wait_recv()
        rdma_y.wait_recv()
        rdma_z.wait_recv()

        @pl.when(mx == 0)
        def _():
            out_ref[pl.ds(sx - 1, 1), :, :] = (
                out_ref[pl.ds(sx - 1, 1), :, :] + hx_ref[:, :][None, :, :]
            )

        @pl.when(mx == 1)
        def _():
            out_ref[pl.ds(0, 1), :, :] = (
                out_ref[pl.ds(0, 1), :, :] + hx_ref[:, :][None, :, :]
            )

        @pl.when(my == 0)
        def _():
            out_ref[:, pl.ds(sy - 1, 1), :] = (
                out_ref[:, pl.ds(sy - 1, 1), :] + hy_ref[:, :][:, None, :]
            )

        @pl.when(my == 1)
        def _():
            out_ref[:, pl.ds(0, 1), :] = (
                out_ref[:, pl.ds(0, 1), :] + hy_ref[:, :][:, None, :]
            )

        @pl.when(mz == 0)
        def _():
            out_ref[:, :, pl.ds(sz - 1, 1)] = (
                out_ref[:, :, pl.ds(sz - 1, 1)] + hz_ref[:, :][:, :, None]
            )

        @pl.when(mz == 1)
        def _():
            out_ref[:, :, pl.ds(0, 1)] = (
                out_ref[:, :, pl.ds(0, 1)] + hz_ref[:, :][:, :, None]
            )

        @pl.when(mx == 0)
        def _():
            out_ref[pl.ds(0, 1), :, :] = jnp.zeros((1, sy, sz), jnp.float32)

        @pl.when(mx == 1)
        def _():
            out_ref[pl.ds(sx - 1, 1), :, :] = jnp.zeros((1, sy, sz), jnp.float32)

        @pl.when(my == 0)
        def _():
            out_ref[:, pl.ds(0, 1), :] = jnp.zeros((sx, 1, sz), jnp.float32)

        @pl.when(my == 1)
        def _():
            out_ref[:, pl.ds(sy - 1, 1), :] = jnp.zeros((sx, 1, sz), jnp.float32)

        @pl.when(mz == 0)
        def _():
            out_ref[:, :, pl.ds(0, 1)] = jnp.zeros((sx, sy, 1), jnp.float32)

        @pl.when(mz == 1)
        def _():
            out_ref[:, :, pl.ds(sz - 1, 1)] = jnp.zeros((sx, sy, 1), jnp.float32)

        rdma_x.wait_send()
        rdma_y.wait_send()
        rdma_z.wait_send()

    return pl.pallas_call(
        body,
        out_shape=jax.ShapeDtypeStruct((sx, sy, sz), jnp.float32),
        in_specs=[pl.BlockSpec(memory_space=pltpu.VMEM)],
        out_specs=pl.BlockSpec(memory_space=pltpu.VMEM),
        scratch_shapes=[
            pltpu.VMEM((sy, sz), jnp.float32),
            pltpu.VMEM((sx, sz), jnp.float32),
            pltpu.VMEM((sx, sy), jnp.float32),
            pltpu.VMEM((sy, sz), jnp.float32),
            pltpu.VMEM((sx, sz), jnp.float32),
            pltpu.VMEM((sx, sy), jnp.float32),
            pltpu.SemaphoreType.DMA((3,)),
            pltpu.SemaphoreType.DMA((3,)),
        ],
        compiler_params=pltpu.CompilerParams(collective_id=0),
    )(u)


# device time: 9918 ns/iter; 1.1318x vs baseline; 1.0823x over previous
import jax
import jax.numpy as jnp
from jax import lax
from jax.experimental import pallas as pl
from jax.experimental.pallas import tpu as pltpu


def kernel(u):
    sx, sy, sz = u.shape
    half = sx // 2

    def body(
        u_ref, out_ref,
        fx_ref, fy_ref, fz_ref,
        hx_ref, hy_ref, hz_ref,
        send_sems, recv_sems,
    ):
        mx = lax.axis_index("x")
        my = lax.axis_index("y")
        mz = lax.axis_index("z")

        nbr_x = (1 - mx, my, mz)
        nbr_y = (mx, 1 - my, mz)
        nbr_z = (mx, my, 1 - mz)

        barrier = pltpu.get_barrier_semaphore()
        for nbr in (nbr_x, nbr_y, nbr_z):
            pl.semaphore_signal(
                barrier, inc=1, device_id=nbr,
                device_id_type=pl.DeviceIdType.MESH,
            )

        x = u_ref[:, :, :]

        bx = mx * (sx - 1)
        by = my * (sy - 1)
        bz = mz * (sz - 1)

        iy_x = lax.broadcasted_iota(jnp.int32, (sy, sz), 0)
        iz_x = lax.broadcasted_iota(jnp.int32, (sy, sz), 1)
        ix_y = lax.broadcasted_iota(jnp.int32, (sx, sz), 0)
        iz_y = lax.broadcasted_iota(jnp.int32, (sx, sz), 1)
        ix_z = lax.broadcasted_iota(jnp.int32, (sx, sy), 0)
        iy_z = lax.broadcasted_iota(jnp.int32, (sx, sy), 1)

        @pl.when(mx == 0)
        def _():
            fx_ref[:, :] = jnp.where(
                (iy_x == by) | (iz_x == bz), 0.0, x[sx - 1, :, :]
            )

        @pl.when(mx == 1)
        def _():
            fx_ref[:, :] = jnp.where(
                (iy_x == by) | (iz_x == bz), 0.0, x[0, :, :]
            )

        @pl.when(my == 0)
        def _():
            fy_ref[:, :] = jnp.where(
                (ix_y == bx) | (iz_y == bz), 0.0, x[:, sy - 1, :]
            )

        @pl.when(my == 1)
        def _():
            fy_ref[:, :] = jnp.where(
                (ix_y == bx) | (iz_y == bz), 0.0, x[:, 0, :]
            )

        @pl.when(mz == 0)
        def _():
            fz_ref[:, :] = jnp.where(
                (ix_z == bx) | (iy_z == by), 0.0, x[:, :, sz - 1]
            )

        @pl.when(mz == 1)
        def _():
            fz_ref[:, :] = jnp.where(
                (ix_z == bx) | (iy_z == by), 0.0, x[:, :, 0]
            )

        def stencil_slab(a, b):
            h = b - a
            xs = x[a:b]
            if a == 0:
                um = jnp.concatenate(
                    [jnp.zeros((1, sy, sz), jnp.float32), x[: b - 1]], axis=0
                )
            else:
                um = x[a - 1 : b - 1]
            if b == sx:
                up = jnp.concatenate(
                    [x[a + 1 :], jnp.zeros((1, sy, sz), jnp.float32)], axis=0
                )
            else:
                up = x[a + 1 : b + 1]
            zy = jnp.zeros((h, 1, sz), jnp.float32)
            zz = jnp.zeros((h, sy, 1), jnp.float32)
            um_y = jnp.concatenate([zy, xs[:, :-1, :]], axis=1)
            up_y = jnp.concatenate([xs[:, 1:, :], zy], axis=1)
            um_z = jnp.concatenate([zz, xs[:, :, :-1]], axis=2)
            up_z = jnp.concatenate([xs[:, :, 1:], zz], axis=2)
            v = um + up + um_y + up_y + um_z + up_z - 6.0 * xs
            ix3 = a + lax.broadcasted_iota(jnp.int32, (h, sy, sz), 0)
            iy3 = lax.broadcasted_iota(jnp.int32, (h, sy, sz), 1)
            iz3 = lax.broadcasted_iota(jnp.int32, (h, sy, sz), 2)
            return jnp.where(
                (ix3 == bx) | (iy3 == by) | (iz3 == bz), 0.0, v
            )

        out_ref[pl.ds(0, half), :, :] = stencil_slab(0, half)

        pl.semaphore_wait(barrier, 3)

        rdma_x = pltpu.make_async_remote_copy(
            src_ref=fx_ref, dst_ref=hx_ref,
            send_sem=send_sems.at[0], recv_sem=recv_sems.at[0],
            device_id=nbr_x, device_id_type=pl.DeviceIdType.MESH,
        )
        rdma_y = pltpu.make_async_remote_copy(
            src_ref=fy_ref, dst_ref=hy_ref,
            send_sem=send_sems.at[1], recv_sem=recv_sems.at[1],
            device_id=nbr_y, device_id_type=pl.DeviceIdType.MESH,
        )
        rdma_z = pltpu.make_async_remote_copy(
            src_ref=fz_ref, dst_ref=hz_ref,
            send_sem=send_sems.at[2], recv_sem=recv_sems.at[2],
            device_id=nbr_z, device_id_type=pl.DeviceIdType.MESH,
        )
        rdma_x.start()
        rdma_y.start()
        rdma_z.start()

        out_ref[pl.ds(half, sx - half), :, :] = stencil_slab(half, sx)

        rdma_x.wait_recv()

        @pl.when(mx == 0)
        def _():
            out_ref[pl.ds(sx - 1, 1), :, :] = (
                out_ref[pl.ds(sx - 1, 1), :, :] + hx_ref[:, :][None, :, :]
            )

        @pl.when(mx == 1)
        def _():
            out_ref[pl.ds(0, 1), :, :] = (
                out_ref[pl.ds(0, 1), :, :] + hx_ref[:, :][None, :, :]
            )

        rdma_y.wait_recv()

        @pl.when(my == 0)
        def _():
            out_ref[:, pl.ds(sy - 1, 1), :] = (
                out_ref[:, pl.ds(sy - 1, 1), :] + hy_ref[:, :][:, None, :]
            )

        @pl.when(my == 1)
        def _():
            out_ref[:, pl.ds(0, 1), :] = (
                out_ref[:, pl.ds(0, 1), :] + hy_ref[:, :][:, None, :]
            )

        rdma_z.wait_recv()

        @pl.when(mz == 0)
        def _():
            out_ref[:, :, pl.ds(sz - 1, 1)] = (
                out_ref[:, :, pl.ds(sz - 1, 1)] + hz_ref[:, :][:, :, None]
            )

        @pl.when(mz == 1)
        def _():
            out_ref[:, :, pl.ds(0, 1)] = (
                out_ref[:, :, pl.ds(0, 1)] + hz_ref[:, :][:, :, None]
            )

        rdma_x.wait_send()
        rdma_y.wait_send()
        rdma_z.wait_send()

    return pl.pallas_call(
        body,
        out_shape=jax.ShapeDtypeStruct((sx, sy, sz), jnp.float32),
        in_specs=[pl.BlockSpec(memory_space=pltpu.VMEM)],
        out_specs=pl.BlockSpec(memory_space=pltpu.VMEM),
        scratch_shapes=[
            pltpu.VMEM((sy, sz), jnp.float32),
            pltpu.VMEM((sx, sz), jnp.float32),
            pltpu.VMEM((sx, sy), jnp.float32),
            pltpu.VMEM((sy, sz), jnp.float32),
            pltpu.VMEM((sx, sz), jnp.float32),
            pltpu.VMEM((sx, sy), jnp.float32),
            pltpu.SemaphoreType.DMA((3,)),
            pltpu.SemaphoreType.DMA((3,)),
        ],
        compiler_params=pltpu.CompilerParams(collective_id=0),
    )(u)
